# baseline (speedup 1.0000x reference)
import functools

import jax
import jax.numpy as jnp
from jax import lax
from jax.experimental import pallas as pl
from jax.experimental.pallas import tpu as pltpu
from jax.experimental.pallas import tpu_sc as plsc

B = 16384
D = 16
NC = 2
NS = 16
NW = NC * NS
B_PER_W = B // NW
CHUNK = 128
N_CHUNKS = B_PER_W // CHUNK

_MESH = plsc.VectorSubcoreMesh(
    core_axis_name="c", subcore_axis_name="s", num_cores=NC, num_subcores=NS
)


@functools.partial(
    pl.kernel,
    out_type=(
        jax.ShapeDtypeStruct((B, D), jnp.float32),
        jax.ShapeDtypeStruct((B, D), jnp.float32),
        jax.ShapeDtypeStruct((B, D), jnp.float32),
    ),
    mesh=_MESH,
    compiler_params=pltpu.CompilerParams(use_tc_tiling_on_sc=False),
    scratch_types=[
        pltpu.VMEM((B_PER_W,), jnp.int32),
        pltpu.VMEM((B_PER_W,), jnp.int32),
        pltpu.VMEM((B_PER_W,), jnp.int32),
        pltpu.VMEM((B_PER_W, D), jnp.float32),
        pltpu.VMEM((B_PER_W, D), jnp.float32),
        pltpu.VMEM((B_PER_W, D), jnp.float32),
        pltpu.SemaphoreType.DMA,
    ],
)
def _sc_gather(
    skill_t, loc_t, role_t, s_idx, l_idx, r_idx,
    out_s, out_l, out_r,
    idx_s, idx_l, idx_r, rows_s, rows_l, rows_r, sem,
):
    wid = lax.axis_index("s") * NC + lax.axis_index("c")
    base = wid * B_PER_W
    pltpu.sync_copy(s_idx.at[pl.ds(base, B_PER_W)], idx_s)
    pltpu.sync_copy(l_idx.at[pl.ds(base, B_PER_W)], idx_l)
    pltpu.sync_copy(r_idx.at[pl.ds(base, B_PER_W)], idx_r)
    copies = []
    for tab, idx_v, rows_v in (
        (skill_t, idx_s, rows_s),
        (loc_t, idx_l, rows_l),
        (role_t, idx_r, rows_r),
    ):
        for g in range(N_CHUNKS):
            sl = pl.ds(g * CHUNK, CHUNK)
            copies.append(pltpu.async_copy(tab.at[idx_v.at[sl]], rows_v.at[sl], sem))
    for c in copies:
        c.wait()
    pltpu.sync_copy(rows_s, out_s.at[pl.ds(base, B_PER_W)])
    pltpu.sync_copy(rows_l, out_l.at[pl.ds(base, B_PER_W)])
    pltpu.sync_copy(rows_r, out_r.at[pl.ds(base, B_PER_W)])


W_SLAB = 512
_TIMING_NO_EXTRACT = False
_TIMING_NO_RELAYOUT = False
SKILL_PAD = 1000064
LOC_PAD = 100096
ROLE_PAD = 1024
_SKILL_ALIGNED = (1000000 // 128) * 128
_LOC_ALIGNED = (100000 // 128) * 128
_ROLE_ALIGNED = (1000 // 128) * 128
_SKILL_SLABS = _SKILL_ALIGNED // W_SLAB
_LOC_SLABS = _LOC_ALIGNED // W_SLAB
_SKILL_TAIL = _SKILL_ALIGNED - _SKILL_SLABS * W_SLAB
_LOC_TAIL = _LOC_ALIGNED - _LOC_SLABS * W_SLAB


@functools.partial(
    pl.kernel,
    out_type=(
        jax.ShapeDtypeStruct((SKILL_PAD, D), jnp.float32),
        jax.ShapeDtypeStruct((LOC_PAD, D), jnp.float32),
        jax.ShapeDtypeStruct((ROLE_PAD, D), jnp.float32),
    ),
    mesh=_MESH,
    compiler_params=pltpu.CompilerParams(needs_layout_passes=False),
    scratch_types=[
        pltpu.VMEM((D, W_SLAB), jnp.float32),
        pltpu.VMEM((D, W_SLAB), jnp.float32),
        pltpu.VMEM((W_SLAB, D), jnp.float32),
        pltpu.SemaphoreType.DMA,
        pltpu.SemaphoreType.DMA,
    ],
)
def _sc_relayout(
    ts_t, tl_t, tr_t, tail_s, tail_l, tail_r,
    out_s, out_l, out_r,
    slab_0, slab_1, rows_0, sin_0, sin_1,
):
    wid = lax.axis_index("s") * NC + lax.axis_index("c")
    lane = lax.iota(jnp.int32, 16)
    slabs, sins = (slab_0, slab_1), (sin_0, sin_1)

    def extract(slab_v, rows_v, width):
        def _row(j):
            c = jnp.full((16,), j, dtype=jnp.int32)
            rows_v[j, :] = plsc.load_gather(slab_v, [lane, c])
        if _TIMING_NO_EXTRACT:
            return
        plsc.parallel_loop(0, width, 1, unroll=8)(_row)

    def do_slab(tab_t, out_hbm, c0, width, slab_v, rows_v):
        pltpu.sync_copy(tab_t.at[0, :, pl.ds(c0, width)], slab_v.at[pl.ds(0, 8), pl.ds(0, width)])
        pltpu.sync_copy(tab_t.at[1, :, pl.ds(c0, width)], slab_v.at[pl.ds(8, 8), pl.ds(0, width)])
        extract(slab_v, rows_v, width)
        pltpu.sync_copy(rows_v.at[pl.ds(0, width)], out_hbm.at[pl.ds(c0, width)])

    for tab_t, out_hbm, n_slabs in (
        (ts_t, out_s, _SKILL_SLABS),
        (tl_t, out_l, _LOC_SLABS),
    ):
        n_rounds = (n_slabs + NW - 1) // NW
        n_pairs = (n_rounds + 1) // 2
        lim = n_slabs * W_SLAB

        def c_of(k):
            return pl.multiple_of((wid + k * NW) * W_SLAB, W_SLAB)

        def start_in(k, b):
            c0 = c_of(k)
            pltpu.async_copy(tab_t.at[0, :, pl.ds(c0, W_SLAB)],
                             slabs[b].at[pl.ds(0, 8)], sins[b])
            pltpu.async_copy(tab_t.at[1, :, pl.ds(c0, W_SLAB)],
                             slabs[b].at[pl.ds(8, 8)], sins[b])

        def wait_in(b):
            pltpu.make_async_copy(tab_t.at[0, :, pl.ds(0, W_SLAB)],
                                  slabs[b].at[pl.ds(0, 8)], sins[b]).wait()
            pltpu.make_async_copy(tab_t.at[1, :, pl.ds(0, W_SLAB)],
                                  slabs[b].at[pl.ds(8, 8)], sins[b]).wait()

        def rnd(k, b):
            @pl.when(c_of(k) < lim)
            def _():
                wait_in(b)

                @pl.when(c_of(k + 1) < lim)
                def _():
                    start_in(k + 1, b ^ 1)

                extract(slabs[b], rows_0, W_SLAB)
                pltpu.sync_copy(rows_0, out_hbm.at[pl.ds(c_of(k), W_SLAB)])

        @pl.when(wid < n_slabs)
        def _():
            start_in(0, 0)

        def _pair(p, carry):
            rnd(2 * p, 0)
            rnd(2 * p + 1, 1)
            return carry

        lax.fori_loop(0, n_pairs, _pair, 0)

    if _SKILL_TAIL:
        @pl.when(wid == 0)
        def _():
            do_slab(ts_t, out_s, _SKILL_SLABS * W_SLAB, _SKILL_TAIL,
                    slab_0, rows_0)

    if _LOC_TAIL:
        @pl.when(wid == 1)
        def _():
            do_slab(tl_t, out_l, _LOC_SLABS * W_SLAB, _LOC_TAIL,
                    slab_0, rows_0)

    @pl.when(wid == 2)
    def _():
        for c0 in range(0, _ROLE_ALIGNED, W_SLAB):
            do_slab(tr_t, out_r, c0, min(W_SLAB, _ROLE_ALIGNED - c0),
                    slab_0, rows_0)

    @pl.when(wid == 3)
    def _():
        for tail, out_hbm, off, n in (
            (tail_s, out_s, _SKILL_ALIGNED, 1000000 - _SKILL_ALIGNED),
            (tail_l, out_l, _LOC_ALIGNED, 100000 - _LOC_ALIGNED),
            (tail_r, out_r, _ROLE_ALIGNED, 1000 - _ROLE_ALIGNED),
        ):
            pltpu.sync_copy(tail, rows_0.at[pl.ds(0, n)])
            pltpu.sync_copy(rows_0.at[pl.ds(0, n)], out_hbm.at[pl.ds(off, n)])


BLK = 2048


def _mlp_body(s_ref, l_ref, r_ref, w1s_ref, w1l_ref, w1r_ref, b1_ref,
              w2_ref, b2_ref, w3_ref, b3_ref, out_ref):
    h = (
        jnp.dot(s_ref[...], w1s_ref[...], preferred_element_type=jnp.float32)
        + jnp.dot(l_ref[...], w1l_ref[...], preferred_element_type=jnp.float32)
        + jnp.dot(r_ref[...], w1r_ref[...], preferred_element_type=jnp.float32)
        + b1_ref[...]
    )
    h = jnp.maximum(h, 0.0)
    h2 = jnp.dot(h, w2_ref[...], preferred_element_type=jnp.float32) + b2_ref[...]
    h2 = jnp.maximum(h2, 0.0)
    logit = jnp.sum(h2 * w3_ref[...], axis=1) + b3_ref[0, 0]
    out_ref[...] = jax.nn.sigmoid(logit)


_mlp = pl.pallas_call(
    _mlp_body,
    grid=(B // BLK,),
    in_specs=[
        pl.BlockSpec((BLK, D), lambda i: (i, 0)),
        pl.BlockSpec((BLK, D), lambda i: (i, 0)),
        pl.BlockSpec((BLK, D), lambda i: (i, 0)),
        pl.BlockSpec((D, 64), lambda i: (0, 0)),
        pl.BlockSpec((D, 64), lambda i: (0, 0)),
        pl.BlockSpec((D, 64), lambda i: (0, 0)),
        pl.BlockSpec((1, 64), lambda i: (0, 0)),
        pl.BlockSpec((64, 32), lambda i: (0, 0)),
        pl.BlockSpec((1, 32), lambda i: (0, 0)),
        pl.BlockSpec((1, 32), lambda i: (0, 0)),
        pl.BlockSpec((1, 1), lambda i: (0, 0), memory_space=pltpu.SMEM),
    ],
    out_specs=pl.BlockSpec((BLK,), lambda i: (i,)),
    out_shape=jax.ShapeDtypeStruct((B,), jnp.float32),
)


def kernel(skill_idx, location_idx, role_idx, skill_table, location_table,
           role_table, W1, b1, W2, b2, W3, b3):
    s_idx = skill_idx.astype(jnp.int32)
    l_idx = location_idx.astype(jnp.int32)
    r_idx = role_idx.astype(jnp.int32)
    if _TIMING_NO_RELAYOUT:
        ts = jnp.zeros((SKILL_PAD, D), jnp.float32)
        tl = jnp.zeros((LOC_PAD, D), jnp.float32)
        tr = jnp.zeros((ROLE_PAD, D), jnp.float32)
    else:
        ts, tl, tr = _sc_relayout(
            skill_table.T.reshape(2, 8, 1000000),
            location_table.T.reshape(2, 8, 100000),
            role_table.T.reshape(2, 8, 1000),
            skill_table[_SKILL_ALIGNED:], location_table[_LOC_ALIGNED:],
            role_table[_ROLE_ALIGNED:])
    es, el, er = _sc_gather(ts, tl, tr, s_idx, l_idx, r_idx)
    return _mlp(
        es, el, er,
        W1[0:D], W1[D:2 * D], W1[2 * D:3 * D],
        b1.reshape(1, 64), W2, b2.reshape(1, 32),
        W3.reshape(1, 32), b3.reshape(1, 1),
    )


# speedup vs baseline: 1.4175x; 1.4175x over previous
import functools

import jax
import jax.numpy as jnp
from jax import lax
from jax.experimental import pallas as pl
from jax.experimental.pallas import tpu as pltpu
from jax.experimental.pallas import tpu_sc as plsc

B = 16384
D = 16
NC = 2
NS = 16
NW = NC * NS
B_PER_W = B // NW
CHUNK = 128
N_CHUNKS = B_PER_W // CHUNK

_MESH = plsc.VectorSubcoreMesh(
    core_axis_name="c", subcore_axis_name="s", num_cores=NC, num_subcores=NS
)


@functools.partial(
    pl.kernel,
    out_type=(
        jax.ShapeDtypeStruct((B, D), jnp.bfloat16),
        jax.ShapeDtypeStruct((B, D), jnp.float32),
        jax.ShapeDtypeStruct((B, D), jnp.float32),
    ),
    mesh=_MESH,
    compiler_params=pltpu.CompilerParams(use_tc_tiling_on_sc=False),
    scratch_types=[
        pltpu.VMEM((B_PER_W,), jnp.int32),
        pltpu.VMEM((B_PER_W,), jnp.int32),
        pltpu.VMEM((B_PER_W,), jnp.int32),
        pltpu.VMEM((B_PER_W, D), jnp.bfloat16),
        pltpu.VMEM((B_PER_W, D), jnp.float32),
        pltpu.VMEM((B_PER_W, D), jnp.float32),
        pltpu.SemaphoreType.DMA,
    ],
)
def _sc_gather(
    skill_t, loc_t, role_t, s_idx, l_idx, r_idx,
    out_s, out_l, out_r,
    idx_s, idx_l, idx_r, rows_s, rows_l, rows_r, sem,
):
    wid = lax.axis_index("s") * NC + lax.axis_index("c")
    base = wid * B_PER_W
    pltpu.sync_copy(s_idx.at[pl.ds(base, B_PER_W)], idx_s)
    pltpu.sync_copy(l_idx.at[pl.ds(base, B_PER_W)], idx_l)
    pltpu.sync_copy(r_idx.at[pl.ds(base, B_PER_W)], idx_r)
    copies = []
    for tab, idx_v, rows_v in (
        (skill_t, idx_s, rows_s),
        (loc_t, idx_l, rows_l),
        (role_t, idx_r, rows_r),
    ):
        for g in range(N_CHUNKS):
            sl = pl.ds(g * CHUNK, CHUNK)
            copies.append(pltpu.async_copy(tab.at[idx_v.at[sl]], rows_v.at[sl], sem))
    for c in copies:
        c.wait()
    pltpu.sync_copy(rows_s, out_s.at[pl.ds(base, B_PER_W)])
    pltpu.sync_copy(rows_l, out_l.at[pl.ds(base, B_PER_W)])
    pltpu.sync_copy(rows_r, out_r.at[pl.ds(base, B_PER_W)])


W_SLAB = 512
_TIMING_NO_EXTRACT = False
_TIMING_NO_RELAYOUT = False
SKILL_PAD = 1000064
LOC_PAD = 100096
ROLE_PAD = 1024
_SKILL_ALIGNED = (1000000 // 128) * 128
_LOC_ALIGNED = (100000 // 128) * 128
_ROLE_ALIGNED = (1000 // 128) * 128
_SKILL_SLABS = _SKILL_ALIGNED // W_SLAB
_LOC_SLABS = _LOC_ALIGNED // W_SLAB
_SKILL_TAIL = _SKILL_ALIGNED - _SKILL_SLABS * W_SLAB
_LOC_TAIL = _LOC_ALIGNED - _LOC_SLABS * W_SLAB


@functools.partial(
    pl.kernel,
    out_type=(
        jax.ShapeDtypeStruct((SKILL_PAD, D), jnp.float32),
        jax.ShapeDtypeStruct((LOC_PAD, D), jnp.float32),
        jax.ShapeDtypeStruct((ROLE_PAD, D), jnp.float32),
    ),
    mesh=_MESH,
    compiler_params=pltpu.CompilerParams(needs_layout_passes=False),
    scratch_types=[
        pltpu.VMEM((D, W_SLAB), jnp.float32),
        pltpu.VMEM((D, W_SLAB), jnp.float32),
        pltpu.VMEM((W_SLAB, D), jnp.float32),
        pltpu.SemaphoreType.DMA,
        pltpu.SemaphoreType.DMA,
    ],
)
def _sc_relayout(
    ts_t, tl_t, tr_t, tail_s, tail_l, tail_r,
    out_s, out_l, out_r,
    slab_0, slab_1, rows_0, sin_0, sin_1,
):
    wid = lax.axis_index("s") * NC + lax.axis_index("c")
    lane = lax.iota(jnp.int32, 16)
    slabs, sins = (slab_0, slab_1), (sin_0, sin_1)

    def extract(slab_v, rows_v, width):
        def _row(j):
            c = jnp.full((16,), j, dtype=jnp.int32)
            rows_v[j, :] = plsc.load_gather(slab_v, [lane, c])
        if _TIMING_NO_EXTRACT:
            return
        plsc.parallel_loop(0, width, 1, unroll=8)(_row)

    def do_slab(tab_t, out_hbm, c0, width, slab_v, rows_v):
        pltpu.sync_copy(tab_t.at[0, :, pl.ds(c0, width)], slab_v.at[pl.ds(0, 8), pl.ds(0, width)])
        pltpu.sync_copy(tab_t.at[1, :, pl.ds(c0, width)], slab_v.at[pl.ds(8, 8), pl.ds(0, width)])
        extract(slab_v, rows_v, width)
        pltpu.sync_copy(rows_v.at[pl.ds(0, width)], out_hbm.at[pl.ds(c0, width)])

    for tab_t, out_hbm, n_slabs in (
        (ts_t, out_s, _SKILL_SLABS),
        (tl_t, out_l, _LOC_SLABS),
    ):
        n_rounds = (n_slabs + NW - 1) // NW
        n_pairs = (n_rounds + 1) // 2
        lim = n_slabs * W_SLAB

        def c_of(k):
            return pl.multiple_of((wid + k * NW) * W_SLAB, W_SLAB)

        def start_in(k, b):
            c0 = c_of(k)
            pltpu.async_copy(tab_t.at[0, :, pl.ds(c0, W_SLAB)],
                             slabs[b].at[pl.ds(0, 8)], sins[b])
            pltpu.async_copy(tab_t.at[1, :, pl.ds(c0, W_SLAB)],
                             slabs[b].at[pl.ds(8, 8)], sins[b])

        def wait_in(b):
            pltpu.make_async_copy(tab_t.at[0, :, pl.ds(0, W_SLAB)],
                                  slabs[b].at[pl.ds(0, 8)], sins[b]).wait()
            pltpu.make_async_copy(tab_t.at[1, :, pl.ds(0, W_SLAB)],
                                  slabs[b].at[pl.ds(8, 8)], sins[b]).wait()

        def rnd(k, b):
            @pl.when(c_of(k) < lim)
            def _():
                wait_in(b)

                @pl.when(c_of(k + 1) < lim)
                def _():
                    start_in(k + 1, b ^ 1)

                extract(slabs[b], rows_0, W_SLAB)
                pltpu.sync_copy(rows_0, out_hbm.at[pl.ds(c_of(k), W_SLAB)])

        @pl.when(wid < n_slabs)
        def _():
            start_in(0, 0)

        def _pair(p, carry):
            rnd(2 * p, 0)
            rnd(2 * p + 1, 1)
            return carry

        lax.fori_loop(0, n_pairs, _pair, 0)

    if _SKILL_TAIL:
        @pl.when(wid == 0)
        def _():
            do_slab(ts_t, out_s, _SKILL_SLABS * W_SLAB, _SKILL_TAIL,
                    slab_0, rows_0)

    if _LOC_TAIL:
        @pl.when(wid == 1)
        def _():
            do_slab(tl_t, out_l, _LOC_SLABS * W_SLAB, _LOC_TAIL,
                    slab_0, rows_0)

    @pl.when(wid == 2)
    def _():
        for c0 in range(0, _ROLE_ALIGNED, W_SLAB):
            do_slab(tr_t, out_r, c0, min(W_SLAB, _ROLE_ALIGNED - c0),
                    slab_0, rows_0)

    @pl.when(wid == 3)
    def _():
        for tail, out_hbm, off, n in (
            (tail_s, out_s, _SKILL_ALIGNED, 1000000 - _SKILL_ALIGNED),
            (tail_l, out_l, _LOC_ALIGNED, 100000 - _LOC_ALIGNED),
            (tail_r, out_r, _ROLE_ALIGNED, 1000 - _ROLE_ALIGNED),
        ):
            pltpu.sync_copy(tail, rows_0.at[pl.ds(0, n)])
            pltpu.sync_copy(rows_0.at[pl.ds(0, n)], out_hbm.at[pl.ds(off, n)])


BLK = 2048


def _mlp_body(s_ref, l_ref, r_ref, w1s_ref, w1l_ref, w1r_ref, b1_ref,
              w2_ref, b2_ref, w3_ref, b3_ref, out_ref):
    h = (
        jnp.dot(s_ref[...].astype(jnp.float32), w1s_ref[...],
                preferred_element_type=jnp.float32)
        + jnp.dot(l_ref[...], w1l_ref[...], preferred_element_type=jnp.float32)
        + jnp.dot(r_ref[...], w1r_ref[...], preferred_element_type=jnp.float32)
        + b1_ref[...]
    )
    h = jnp.maximum(h, 0.0)
    h2 = jnp.dot(h, w2_ref[...], preferred_element_type=jnp.float32) + b2_ref[...]
    h2 = jnp.maximum(h2, 0.0)
    logit = jnp.sum(h2 * w3_ref[...], axis=1) + b3_ref[0, 0]
    out_ref[...] = jax.nn.sigmoid(logit)


_mlp = pl.pallas_call(
    _mlp_body,
    grid=(B // BLK,),
    in_specs=[
        pl.BlockSpec((BLK, D), lambda i: (i, 0)),
        pl.BlockSpec((BLK, D), lambda i: (i, 0)),
        pl.BlockSpec((BLK, D), lambda i: (i, 0)),
        pl.BlockSpec((D, 64), lambda i: (0, 0)),
        pl.BlockSpec((D, 64), lambda i: (0, 0)),
        pl.BlockSpec((D, 64), lambda i: (0, 0)),
        pl.BlockSpec((1, 64), lambda i: (0, 0)),
        pl.BlockSpec((64, 32), lambda i: (0, 0)),
        pl.BlockSpec((1, 32), lambda i: (0, 0)),
        pl.BlockSpec((1, 32), lambda i: (0, 0)),
        pl.BlockSpec((1, 1), lambda i: (0, 0), memory_space=pltpu.SMEM),
    ],
    out_specs=pl.BlockSpec((BLK,), lambda i: (i,)),
    out_shape=jax.ShapeDtypeStruct((B,), jnp.float32),
)


def kernel(skill_idx, location_idx, role_idx, skill_table, location_table,
           role_table, W1, b1, W2, b2, W3, b3):
    s_idx = skill_idx.astype(jnp.int32)
    l_idx = location_idx.astype(jnp.int32)
    r_idx = role_idx.astype(jnp.int32)
    es, el, er = _sc_gather(skill_table.astype(jnp.bfloat16),
                            location_table, role_table,
                            s_idx, l_idx, r_idx)
    return _mlp(
        es, el, er,
        W1[0:D], W1[D:2 * D], W1[2 * D:3 * D],
        b1.reshape(1, 64), W2, b2.reshape(1, 32),
        W3.reshape(1, 32), b3.reshape(1, 1),
    )


# speedup vs baseline: 1.5990x; 1.1280x over previous
import functools

import jax
import jax.numpy as jnp
from jax import lax
from jax.experimental import pallas as pl
from jax.experimental.pallas import tpu as pltpu
from jax.experimental.pallas import tpu_sc as plsc

B = 16384
D = 16
NC = 2
NS = 16
NW = NC * NS
B_PER_W = B // NW
CHUNK = 128
N_CHUNKS = B_PER_W // CHUNK

_MESH = plsc.VectorSubcoreMesh(
    core_axis_name="c", subcore_axis_name="s", num_cores=NC, num_subcores=NS
)


@functools.partial(
    pl.kernel,
    out_type=(
        jax.ShapeDtypeStruct((B, D), jnp.float32),
        jax.ShapeDtypeStruct((B, D), jnp.float32),
        jax.ShapeDtypeStruct((B, D), jnp.float32),
    ),
    mesh=_MESH,
    compiler_params=pltpu.CompilerParams(use_tc_tiling_on_sc=False),
    scratch_types=[
        pltpu.VMEM((B_PER_W,), jnp.int32),
        pltpu.VMEM((B_PER_W,), jnp.int32),
        pltpu.VMEM((B_PER_W,), jnp.int32),
        pltpu.VMEM((B_PER_W, D), jnp.float32),
        pltpu.VMEM((B_PER_W, D), jnp.float32),
        pltpu.VMEM((B_PER_W, D), jnp.float32),
        pltpu.SemaphoreType.DMA,
    ],
)
def _sc_gather(
    skill_t, loc_t, role_t, s_idx, l_idx, r_idx,
    out_s, out_l, out_r,
    idx_s, idx_l, idx_r, rows_s, rows_l, rows_r, sem,
):
    wid = lax.axis_index("s") * NC + lax.axis_index("c")
    base = wid * B_PER_W
    pltpu.sync_copy(s_idx.at[pl.ds(base, B_PER_W)], idx_s)
    pltpu.sync_copy(l_idx.at[pl.ds(base, B_PER_W)], idx_l)
    pltpu.sync_copy(r_idx.at[pl.ds(base, B_PER_W)], idx_r)
    copies = []
    for tab, idx_v, rows_v in (
        (skill_t, idx_s, rows_s),
        (loc_t, idx_l, rows_l),
        (role_t, idx_r, rows_r),
    ):
        for g in range(N_CHUNKS):
            sl = pl.ds(g * CHUNK, CHUNK)
            copies.append(pltpu.async_copy(tab.at[idx_v.at[sl]], rows_v.at[sl], sem))
    for c in copies:
        c.wait()
    pltpu.sync_copy(rows_s, out_s.at[pl.ds(base, B_PER_W)])
    pltpu.sync_copy(rows_l, out_l.at[pl.ds(base, B_PER_W)])
    pltpu.sync_copy(rows_r, out_r.at[pl.ds(base, B_PER_W)])


BLK = 2048


def _mlp_body(s_ref, l_ref, r_ref, w1s_ref, w1l_ref, w1r_ref, b1_ref,
              w2_ref, b2_ref, w3_ref, b3_ref, out_ref):
    h = (
        jnp.dot(s_ref[...], w1s_ref[...], preferred_element_type=jnp.float32)
        + jnp.dot(l_ref[...], w1l_ref[...], preferred_element_type=jnp.float32)
        + jnp.dot(r_ref[...], w1r_ref[...], preferred_element_type=jnp.float32)
        + b1_ref[...]
    )
    h = jnp.maximum(h, 0.0)
    h2 = jnp.dot(h, w2_ref[...], preferred_element_type=jnp.float32) + b2_ref[...]
    h2 = jnp.maximum(h2, 0.0)
    logit = jnp.sum(h2 * w3_ref[...], axis=1) + b3_ref[0, 0]
    out_ref[...] = jax.nn.sigmoid(logit)


_mlp = pl.pallas_call(
    _mlp_body,
    grid=(B // BLK,),
    in_specs=[
        pl.BlockSpec((BLK, D), lambda i: (i, 0)),
        pl.BlockSpec((BLK, D), lambda i: (i, 0)),
        pl.BlockSpec((BLK, D), lambda i: (i, 0)),
        pl.BlockSpec((D, 64), lambda i: (0, 0)),
        pl.BlockSpec((D, 64), lambda i: (0, 0)),
        pl.BlockSpec((D, 64), lambda i: (0, 0)),
        pl.BlockSpec((1, 64), lambda i: (0, 0)),
        pl.BlockSpec((64, 32), lambda i: (0, 0)),
        pl.BlockSpec((1, 32), lambda i: (0, 0)),
        pl.BlockSpec((1, 32), lambda i: (0, 0)),
        pl.BlockSpec((1, 1), lambda i: (0, 0), memory_space=pltpu.SMEM),
    ],
    out_specs=pl.BlockSpec((BLK,), lambda i: (i,)),
    out_shape=jax.ShapeDtypeStruct((B,), jnp.float32),
)


def kernel(skill_idx, location_idx, role_idx, skill_table, location_table,
           role_table, W1, b1, W2, b2, W3, b3):
    s_idx = skill_idx.astype(jnp.int32)
    l_idx = location_idx.astype(jnp.int32)
    r_idx = role_idx.astype(jnp.int32)
    es, el, er = _sc_gather(skill_table, location_table, role_table,
                            s_idx, l_idx, r_idx)
    return _mlp(
        es, el, er,
        W1[0:D], W1[D:2 * D], W1[2 * D:3 * D],
        b1.reshape(1, 64), W2, b2.reshape(1, 32),
        W3.reshape(1, 32), b3.reshape(1, 1),
    )
